# baseline (speedup 1.0000x reference)
import functools

import jax
import jax.numpy as jnp
from jax import lax
from jax.experimental import pallas as pl
from jax.experimental.pallas import tpu as pltpu
from jax.experimental.pallas import tpu_sc as plsc

NUM_EXPERTS = 64
TOP_K = 8
MODEL_DIM = 2048
T = 16384
BLOCK_T = 1024

NC = 2
NS = 16
NW = NC * NS
TPW = T // NW
GROUPS = TPW // 16

INT_MIN = -(2 ** 31)


def _tc_block(x_ref, w_ref, probs_ref, keys_ref):
    x = x_ref[...]
    w = w_ref[...]
    logits_t = lax.dot_general(
        w, x,
        dimension_numbers=(((1,), (1,)), ((), ())),
        preferred_element_type=jnp.float32,
    )
    m = jnp.max(logits_t, axis=0, keepdims=True)
    e = jnp.exp(logits_t - m)
    s = jnp.sum(e, axis=0, keepdims=True)
    probs_ref[...] = (e * (1.0 / s)).T
    keys_ref[...] = lax.bitcast_convert_type(e, jnp.int32)


def _sc_topk(keys_hbm, tw_hbm, ti_hbm, keys_v, tw_v, ti_v):
    wid = lax.axis_index("s") * NC + lax.axis_index("c")
    base = wid * TPW
    pltpu.sync_copy(keys_hbm.at[:, pl.ds(base, TPW)], keys_v)

    @plsc.parallel_loop(0, GROUPS, 1, unroll=2)
    def group(g):
        t = [jnp.full((16,), INT_MIN, jnp.int32) for _ in range(TOP_K)]
        ti = [jnp.full((16,), 0, jnp.int32) for _ in range(TOP_K)]
        for ex in range(NUM_EXPERTS):
            v = keys_v[ex, pl.ds(g * 16, 16)]
            vi = jnp.full((16,), ex, jnp.int32)
            for j in range(TOP_K):
                c = v > t[j]
                hi = jnp.where(c, v, t[j])
                v = jnp.where(c, t[j], v)
                t[j] = hi
                hii = jnp.where(c, vi, ti[j])
                vi = jnp.where(c, ti[j], vi)
                ti[j] = hii
        vals = [lax.bitcast_convert_type(tj, jnp.float32) for tj in t]
        ssum = vals[0]
        for vv in vals[1:]:
            ssum = ssum + vv
        inv = 1.0 / ssum
        for j in range(TOP_K):
            tw_v[j, pl.ds(g * 16, 16)] = vals[j] * inv
            ti_v[j, pl.ds(g * 16, 16)] = ti[j]
    pltpu.sync_copy(tw_v, tw_hbm.at[:, pl.ds(base, TPW)])
    pltpu.sync_copy(ti_v, ti_hbm.at[:, pl.ds(base, TPW)])


@functools.partial(jax.jit, static_argnames=("interpret",))
def _run(hidden_states, weight, interpret=False):
    x = hidden_states.reshape(-1, MODEL_DIM)
    grid = (T // BLOCK_T,)
    probs, keys_t = pl.pallas_call(
        _tc_block,
        grid=grid,
        in_specs=[
            pl.BlockSpec((BLOCK_T, MODEL_DIM), lambda i: (i, 0)),
            pl.BlockSpec((NUM_EXPERTS, MODEL_DIM), lambda i: (0, 0)),
        ],
        out_specs=[
            pl.BlockSpec((BLOCK_T, NUM_EXPERTS), lambda i: (i, 0)),
            pl.BlockSpec((NUM_EXPERTS, BLOCK_T), lambda i: (0, i)),
        ],
        out_shape=[
            jax.ShapeDtypeStruct((T, NUM_EXPERTS), jnp.float32),
            jax.ShapeDtypeStruct((NUM_EXPERTS, T), jnp.int32),
        ],
        interpret=interpret,
    )(x, weight)

    sc_call = pl.kernel(
        _sc_topk,
        out_type=[
            jax.ShapeDtypeStruct((TOP_K, T), jnp.float32),
            jax.ShapeDtypeStruct((TOP_K, T), jnp.int32),
        ],
        mesh=plsc.VectorSubcoreMesh(core_axis_name="c", subcore_axis_name="s"),
        scratch_types=[
            pltpu.VMEM((NUM_EXPERTS, TPW), jnp.int32),
            pltpu.VMEM((TOP_K, TPW), jnp.float32),
            pltpu.VMEM((TOP_K, TPW), jnp.int32),
        ],
        interpret=interpret,
    )
    tw_t, ti_t = sc_call(keys_t)
    return probs, tw_t.T, ti_t.T


def kernel(hidden_states, weight):
    return _run(hidden_states, weight)


# speedup vs baseline: 1.9877x; 1.9877x over previous
import functools

import jax
import jax.numpy as jnp
from jax import lax
from jax.experimental import pallas as pl
from jax.experimental.pallas import tpu as pltpu

NUM_EXPERTS = 64
TOP_K = 8
MODEL_DIM = 2048
T = 16384
BLOCK_T = 1024


def _router_block(x_ref, w_ref, probs_ref, tw_ref, ti_ref):
    x = x_ref[...]
    w = w_ref[...]
    logits_t = lax.dot_general(
        w, x,
        dimension_numbers=(((1,), (1,)), ((), ())),
        preferred_element_type=jnp.float32,
    )
    m = jnp.max(logits_t, axis=0, keepdims=True)
    e = jnp.exp(logits_t - m)
    s = jnp.sum(e, axis=0, keepdims=True)
    probs_ref[...] = (e * (1.0 / s)).T

    iota_e = lax.broadcasted_iota(jnp.int32, e.shape, 0)
    p = e
    vrows, irows = [], []
    for _ in range(TOP_K):
        cur = jnp.max(p, axis=0, keepdims=True)
        idx = jnp.min(jnp.where(p == cur, iota_e, NUM_EXPERTS), axis=0,
                      keepdims=True)
        vrows.append(cur)
        irows.append(idx)
        p = jnp.where(iota_e == idx, -1.0, p)
    vals = jnp.concatenate(vrows, axis=0)
    tw_ref[...] = vals * (1.0 / jnp.sum(vals, axis=0, keepdims=True))
    ti_ref[...] = jnp.concatenate(irows, axis=0)


@functools.partial(jax.jit, static_argnames=("interpret",))
def _run(hidden_states, weight, interpret=False):
    x = hidden_states.reshape(-1, MODEL_DIM)
    grid = (T // BLOCK_T,)
    probs, tw_t, ti_t = pl.pallas_call(
        _router_block,
        grid=grid,
        in_specs=[
            pl.BlockSpec((BLOCK_T, MODEL_DIM), lambda i: (i, 0)),
            pl.BlockSpec((NUM_EXPERTS, MODEL_DIM), lambda i: (0, 0)),
        ],
        out_specs=[
            pl.BlockSpec((BLOCK_T, NUM_EXPERTS), lambda i: (i, 0)),
            pl.BlockSpec((TOP_K, BLOCK_T), lambda i: (0, i)),
            pl.BlockSpec((TOP_K, BLOCK_T), lambda i: (0, i)),
        ],
        out_shape=[
            jax.ShapeDtypeStruct((T, NUM_EXPERTS), jnp.float32),
            jax.ShapeDtypeStruct((TOP_K, T), jnp.float32),
            jax.ShapeDtypeStruct((TOP_K, T), jnp.int32),
        ],
        interpret=interpret,
    )(x, weight)
    return probs, tw_t.T, ti_t.T


def kernel(hidden_states, weight):
    return _run(hidden_states, weight)
